# baseline (speedup 1.0000x reference)
import functools

import jax
import jax.numpy as jnp
from jax import lax
from jax.experimental import pallas as pl
from jax.experimental.pallas import tpu as pltpu
from jax.experimental.pallas import tpu_sc as plsc

N = 100000
E = 3200000
DIM = 8
DEPTH = 8
NUM_DAGS = 128
NEXEC = 50

NC = 2
NS = 16
NW = NC * NS
EPW = 102400
E_PAD = NW * EPW
CHUNK = 1024
NVEC = CHUNK // 16
SUB = 1024
MAXSUB = CHUNK // SUB
NPAIR = EPW // (2 * CHUNK)
R_TAB = 100352
SLAB = R_TAB // NS
TRASH = R_TAB - 8

_mesh = plsc.VectorSubcoreMesh(
    core_axis_name="c", subcore_axis_name="s", num_cores=NC, num_subcores=NS)


@functools.partial(
    pl.kernel,
    out_type=(
        jax.ShapeDtypeStruct((NC, R_TAB, DIM), jnp.float32),
        jax.ShapeDtypeStruct((NC, R_TAB), jnp.float32),
        jax.ShapeDtypeStruct((NUM_DAGS, 8), jnp.float32),
    ),
    mesh=_mesh,
    compiler_params=pltpu.CompilerParams(use_tc_tiling_on_sc=False,
                                         needs_layout_passes=False),
    scratch_types=[
        pltpu.VMEM_SHARED((R_TAB, DIM), jnp.float32),
        pltpu.VMEM_SHARED((R_TAB,), jnp.float32),
        pltpu.VMEM((CHUNK,), jnp.int32),
        pltpu.VMEM((CHUNK,), jnp.int32),
        pltpu.VMEM((CHUNK,), jnp.float32),
        pltpu.VMEM((CHUNK,), jnp.int32),
        pltpu.VMEM((CHUNK,), jnp.int32),
        pltpu.VMEM((CHUNK,), jnp.float32),
        pltpu.VMEM((CHUNK + SUB,), jnp.int32),
        pltpu.VMEM((CHUNK + SUB,), jnp.int32),
        pltpu.VMEM((MAXSUB, SUB), jnp.int32),
        pltpu.VMEM((MAXSUB, SUB), jnp.int32),
        pltpu.VMEM((CHUNK, DIM), jnp.float32),
        pltpu.VMEM((CHUNK, DIM), jnp.float32),
        pltpu.VMEM((SUB,), jnp.float32),
        pltpu.VMEM((NUM_DAGS,), jnp.int32),
        pltpu.VMEM((NUM_DAGS, 8), jnp.float32),
        pltpu.SemaphoreType.DMA,
        pltpu.SemaphoreType.DMA,
        pltpu.SemaphoreType.DMA,
        pltpu.SemaphoreType.DMA,
    ],
)
def _edge_sc(y_hbm, src_hbm, dst_hbm, mask_hbm, z8_hbm, z1_hbm, xp_hbm, ptr_hbm,
             agg_out, cnt_out, dagx_out,
             agg_sh, cnt_sh, src0, dst0, mk0, src1, dst1, mk1, psrc, pdst,
             sidx0, sidx1, rows0, rows1, ones_v, ptr_v, dagx_v,
             isem, gsem, ssem, csem):
    c = lax.axis_index("c")
    s = lax.axis_index("s")

    pltpu.sync_copy(z8_hbm, agg_sh.at[pl.ds(s * SLAB, SLAB)])
    pltpu.sync_copy(z1_hbm, cnt_sh.at[pl.ds(s * SLAB, SLAB)])
    for i in range(SUB // 16):
        ones_v[pl.ds(i * 16, 16)] = jnp.ones((16,), jnp.float32)
    plsc.subcore_barrier()

    base = (c * NS + s) * EPW

    def fire_in(ch, sv, dv, mv):
        off = base + ch * CHUNK
        pltpu.async_copy(src_hbm.at[pl.ds(off, CHUNK)], sv, isem)
        pltpu.async_copy(dst_hbm.at[pl.ds(off, CHUNK)], dv, isem)
        pltpu.async_copy(mask_hbm.at[pl.ds(off, CHUNK)], mv, isem)

    def drain_in(ch, sv, dv, mv):
        off = base + ch * CHUNK
        pltpu.make_async_copy(src_hbm.at[pl.ds(off, CHUNK)], sv, isem).wait()
        pltpu.make_async_copy(dst_hbm.at[pl.ds(off, CHUNK)], dv, isem).wait()
        pltpu.make_async_copy(mask_hbm.at[pl.ds(off, CHUNK)], mv, isem).wait()

    def compress(sv, dv, mv, sidx):
        def comp_i(i, off):
            sl = pl.ds(i * 16, 16)
            m = mv[sl] > 0.0
            mi = m.astype(jnp.int32)
            pos = plsc.cumsum(mi)
            idx = off + pos - mi
            plsc.store_scatter(psrc, [idx], sv[sl], mask=m)
            plsc.store_scatter(pdst, [idx], dv[sl], mask=m)
            return off + pos[15]
        nc_ = lax.fori_loop(0, NVEC, comp_i, 0)
        for j in range(SUB // 16):
            psrc[pl.ds(nc_ + j * 16, 16)] = jnp.full((16,), TRASH, jnp.int32)
            pdst[pl.ds(nc_ + j * 16, 16)] = jnp.zeros((16,), jnp.int32)
        nsub = (nc_ + (SUB - 1)) // SUB
        vpr = SUB // 16
        def cp(k, _):
            sidx[k // vpr, pl.ds((k % vpr) * 16, 16)] = psrc[pl.ds(k * 16, 16)]
            return 0
        lax.fori_loop(0, nsub * vpr, cp, 0)
        return nsub

    def fire_gather(nsub, rows):
        for j in range(MAXSUB):
            @pl.when(j < nsub)
            def _():
                pltpu.async_copy(y_hbm.at[pdst.at[pl.ds(j * SUB, SUB)]],
                                 rows.at[pl.ds(j * SUB, SUB)], gsem)

    def drain_gather(nsub, rows):
        for j in range(MAXSUB):
            @pl.when(j < nsub)
            def _():
                pltpu.make_async_copy(y_hbm.at[pdst.at[pl.ds(j * SUB, SUB)]],
                                      rows.at[pl.ds(j * SUB, SUB)], gsem).wait()

    def fire_scatter(nsub, rows, sidx):
        for j in range(MAXSUB):
            @pl.when(j < nsub)
            def _():
                pltpu.async_copy(rows.at[pl.ds(j * SUB, SUB)],
                                 agg_sh.at[sidx.at[j]], ssem, add=True)
                pltpu.async_copy(ones_v, cnt_sh.at[sidx.at[j]], csem, add=True)

    def drain_scatter(nsub, rows, sidx):
        for j in range(MAXSUB):
            @pl.when(j < nsub)
            def _():
                pltpu.make_async_copy(rows.at[pl.ds(j * SUB, SUB)],
                                      agg_sh.at[sidx.at[j]], ssem).wait()
                pltpu.make_async_copy(ones_v, cnt_sh.at[sidx.at[j]], csem).wait()

    fire_in(0, src0, dst0, mk0)
    fire_in(1, src1, dst1, mk1)

    def pair_body(t, ns_prev):
        drain_in(2 * t, src0, dst0, mk0)
        ns_a = compress(src0, dst0, mk0, sidx0)
        fire_in(2 * t + 2, src0, dst0, mk0)
        fire_gather(ns_a, rows0)
        drain_scatter(ns_prev, rows1, sidx1)
        drain_gather(ns_a, rows0)
        fire_scatter(ns_a, rows0, sidx0)
        drain_in(2 * t + 1, src1, dst1, mk1)
        ns_b = compress(src1, dst1, mk1, sidx1)
        fire_in(2 * t + 3, src1, dst1, mk1)
        fire_gather(ns_b, rows1)
        drain_scatter(ns_a, rows0, sidx0)
        drain_gather(ns_b, rows1)
        fire_scatter(ns_b, rows1, sidx1)
        return ns_b

    ns_last = lax.fori_loop(0, NPAIR, pair_body, 0)
    drain_scatter(ns_last, rows1, sidx1)
    drain_in(NPAIR * 2, src0, dst0, mk0)
    drain_in(NPAIR * 2 + 1, src1, dst1, mk1)

    plsc.subcore_barrier()
    pltpu.sync_copy(agg_sh.at[pl.ds(s * SLAB, SLAB)],
                    agg_out.at[c, pl.ds(s * SLAB, SLAB)])
    pltpu.sync_copy(cnt_sh.at[pl.ds(s * SLAB, SLAB)],
                    cnt_out.at[c, pl.ds(s * SLAB, SLAB)])

    @pl.when((c == 0) & (s == 0))
    def _():
        pltpu.sync_copy(ptr_hbm.at[pl.ds(0, NUM_DAGS)], ptr_v)
        pltpu.async_copy(xp_hbm.at[ptr_v], dagx_v, gsem).wait()
        pltpu.sync_copy(dagx_v, dagx_out)


_B = 2000
_G = N // _B


def _leaky(v):
    return jnp.maximum(v, 0.2 * v)


def _mm(a, w):
    return lax.dot_general(a, w, (((1,), (0,)), ((), ())),
                           preferred_element_type=jnp.float32)


def _mlp3(w, a):
    a = _leaky(_mm(a, w[0]) + w[1])
    a = _leaky(_mm(a, w[2]) + w[3])
    return _mm(a, w[4]) + w[5]


def _rows(m):
    return pl.BlockSpec((_B, m), lambda i: (i, 0))


def _full(shape):
    return pl.BlockSpec(shape, lambda i: (0,) * len(shape))


def _prep_body(x_ref, *refs):
    w = [r[...] for r in refs[:12]]
    h_ref, y_ref = refs[12:]
    h = _mlp3(w[:6], x_ref[...])
    h_ref[...] = h
    y_ref[...] = _mlp3(w[6:], h)


def _upd_body(aggA, aggB, cntA, cntB, h_ref, *refs):
    w = [r[...] for r in refs[:12]]
    hn_ref, yn_ref = refs[12:]
    agg = aggA[0] + aggB[0]
    cnt = cntA[0] + cntB[0]
    u = _mlp3(w[:6], agg)
    hn = h_ref[...] + jnp.where(cnt > 0.0, u, 0.0)
    hn_ref[...] = hn
    yn_ref[...] = _mlp3(w[6:], hn)


def _seg_onehot(plo_ref, phi_ref):
    i = pl.program_id(0)
    rowid = i * _B + lax.broadcasted_iota(jnp.int32, (_B, 1), 0)
    return ((rowid >= plo_ref[...]) & (rowid < phi_ref[...])).astype(jnp.float32)


def _dagsum_body(x_ref, h_ref, plo_ref, phi_ref, *refs):
    w = [r[...] for r in refs[:13]]
    dag_ref, glob_ref = refs[13:]
    i = pl.program_id(0)
    oh = _seg_onehot(plo_ref, phi_ref)
    z = _leaky(_mm(x_ref[...], w[0]) + _mm(h_ref[...], w[1]) + w[2])
    z = _leaky(_mm(z, w[3]) + w[4])
    z = _mm(z, w[5]) + w[6]
    part = lax.dot_general(oh, z, (((0,), (0,)), ((), ())),
                           preferred_element_type=jnp.float32)

    @pl.when(i == 0)
    def _():
        dag_ref[...] = part

    @pl.when(i > 0)
    def _():
        dag_ref[...] = dag_ref[...] + part

    @pl.when(i == _G - 1)
    def _():
        g = _mlp3(w[7:13], dag_ref[...])
        glob_ref[...] = jnp.sum(g, axis=0, keepdims=True)


def _nscore_body(x_ref, h_ref, plo_ref, phi_ref, dsum_ref, glob_ref, *refs):
    w = [r[...] for r in refs[:11]]
    ns_ref = refs[11]
    oh = _seg_onehot(plo_ref, phi_ref)
    drep = lax.dot_general(oh, dsum_ref[...], (((1,), (0,)), ((), ())),
                           preferred_element_type=jnp.float32)
    pre = (_mm(x_ref[...], w[0]) + _mm(h_ref[...], w[1]) + _mm(drep, w[2])
           + _mm(glob_ref[...], w[3]) + w[4])
    a = _leaky(pre)
    a = _leaky(_mm(a, w[5]) + w[6])
    a = _leaky(_mm(a, w[7]) + w[8])
    ns_ref[...] = _mm(a, w[9]) + w[10]


def _dscore_body(dagf_ref, dsum_ref, glob_ref, ea_ref, *refs):
    w = [r[...] for r in refs[:11]]
    out_ref = refs[11]
    m = (_mm(dagf_ref[...], w[0]) + _mm(dsum_ref[...], w[1])
         + _mm(glob_ref[...], w[2]) + w[4])
    e = ea_ref[...] * w[3]
    rows = NUM_DAGS * NEXEC
    q = ((lax.broadcasted_iota(jnp.int32, (rows, NUM_DAGS), 0) // NEXEC)
         == lax.broadcasted_iota(jnp.int32, (rows, NUM_DAGS), 1)).astype(jnp.float32)
    p = ((lax.broadcasted_iota(jnp.int32, (rows, NEXEC), 0) % NEXEC)
         == lax.broadcasted_iota(jnp.int32, (rows, NEXEC), 1)).astype(jnp.float32)
    a = _leaky(_mm(q, m) + _mm(p, e))
    a = _leaky(_mm(a, w[5]) + w[6])
    a = _leaky(_mm(a, w[7]) + w[8])
    out_ref[...] = _mm(a, w[9]) + w[10]


def _flat(layers):
    return [a for W, b in layers for a in (W, b.reshape(1, -1))]


def kernel(x, edge_index, edge_mask_batch, ptr, params):
    f32 = jnp.float32
    epad = E_PAD + CHUNK - E
    src_p = jnp.pad(edge_index[0], (0, epad)).astype(jnp.int32)
    dst_p = jnp.pad(edge_index[1], (0, epad)).astype(jnp.int32)
    maskf = jnp.pad(edge_mask_batch, ((0, 0), (0, epad))).astype(f32)
    xp8 = jnp.pad(x, ((0, 0), (0, 8 - x.shape[1])))
    ptr_i = ptr.astype(jnp.int32)
    z8 = jnp.zeros((SLAB, DIM), f32)
    z1 = jnp.zeros((SLAB,), f32)
    plo = ptr_i[:-1].reshape(1, NUM_DAGS)
    phi = ptr_i[1:].reshape(1, NUM_DAGS)
    ea = (jnp.arange(NEXEC, dtype=f32) / NEXEC).reshape(NEXEC, 1)

    w_prep = _flat(params['node_prep'])
    w_msg = _flat(params['node_msg'])
    w_upd = _flat(params['node_update'])
    w_dagm = _flat(params['dag_msg'])
    w_glob = _flat(params['glob_msg'])
    w_ns = _flat(params['node_score'])
    w_ds = _flat(params['dag_score'])

    wspecs = lambda ws: [_full(w.shape) for w in ws]

    h, y = pl.pallas_call(
        _prep_body,
        grid=(_G,),
        in_specs=[_rows(5)] + wspecs(w_prep + w_msg),
        out_specs=[_rows(DIM), _rows(DIM)],
        out_shape=[jax.ShapeDtypeStruct((N, DIM), f32)] * 2,
    )(x, *w_prep, *w_msg)

    dagx = None
    for d in range(DEPTH):
        agg2, cnt2, dagx = _edge_sc(
            y, src_p, dst_p, maskf[d], z8, z1, xp8, ptr_i)
        cnt3 = cnt2.reshape(NC, R_TAB, 1)
        h, y = pl.pallas_call(
            _upd_body,
            grid=(_G,),
            in_specs=[
                pl.BlockSpec((1, _B, DIM), lambda i: (0, i, 0)),
                pl.BlockSpec((1, _B, DIM), lambda i: (1, i, 0)),
                pl.BlockSpec((1, _B, 1), lambda i: (0, i, 0)),
                pl.BlockSpec((1, _B, 1), lambda i: (1, i, 0)),
                _rows(DIM),
            ] + wspecs(w_upd + w_msg),
            out_specs=[_rows(DIM), _rows(DIM)],
            out_shape=[jax.ShapeDtypeStruct((N, DIM), f32)] * 2,
        )(agg2, agg2, cnt3, cnt3, h, *w_upd, *w_msg)

    w1 = params['dag_msg'][0][0]
    w_dag_split = ([w1[:5], w1[5:], w_dagm[1]] + w_dagm[2:])
    dsum, glob = pl.pallas_call(
        _dagsum_body,
        grid=(_G,),
        in_specs=[_rows(5), _rows(DIM), _full((1, NUM_DAGS)), _full((1, NUM_DAGS))]
        + wspecs(w_dag_split + w_glob),
        out_specs=[_full((NUM_DAGS, DIM)), _full((1, DIM))],
        out_shape=[jax.ShapeDtypeStruct((NUM_DAGS, DIM), f32),
                   jax.ShapeDtypeStruct((1, DIM), f32)],
    )(x, h, plo, phi, *w_dag_split, *w_glob)

    wn1 = params['node_score'][0][0]
    w_ns_split = ([wn1[:5], wn1[5:13], wn1[13:21], wn1[21:29], w_ns[1]]
                  + w_ns[2:])
    ns = pl.pallas_call(
        _nscore_body,
        grid=(_G,),
        in_specs=[_rows(5), _rows(DIM), _full((1, NUM_DAGS)), _full((1, NUM_DAGS)),
                  _full((NUM_DAGS, DIM)), _full((1, DIM))] + wspecs(w_ns_split),
        out_specs=[_rows(1)],
        out_shape=[jax.ShapeDtypeStruct((N, 1), f32)],
    )(x, h, plo, phi, dsum, glob, *w_ns_split)[0]

    wd1 = params['dag_score'][0][0]
    w_ds_split = ([wd1[:3], wd1[3:11], wd1[11:19], wd1[19:20].reshape(1, -1),
                   w_ds[1]] + w_ds[2:])
    dagf = dagx[:, :3]
    dsc = pl.pallas_call(
        _dscore_body,
        grid=(1,),
        in_specs=[_full((NUM_DAGS, 3)), _full((NUM_DAGS, DIM)), _full((1, DIM)),
                  _full((NEXEC, 1))] + wspecs(w_ds_split),
        out_specs=[_full((NUM_DAGS * NEXEC, 1))],
        out_shape=[jax.ShapeDtypeStruct((NUM_DAGS * NEXEC, 1), f32)],
    )(dagf, dsum, glob, ea, *w_ds_split)[0]

    return ns.reshape(N), dsc.reshape(NUM_DAGS, NEXEC)


# speedup vs baseline: 1.0007x; 1.0007x over previous
import functools

import jax
import jax.numpy as jnp
from jax import lax
from jax.experimental import pallas as pl
from jax.experimental.pallas import tpu as pltpu
from jax.experimental.pallas import tpu_sc as plsc

N = 100000
E = 3200000
DIM = 8
DEPTH = 8
NUM_DAGS = 128
NEXEC = 50

NC = 2
NS = 16
NW = NC * NS
EPW = 102400
E_PAD = NW * EPW
CHUNK = 1024
NVEC = CHUNK // 16
SUB = 1024
MAXSUB = CHUNK // SUB
NPAIR = EPW // (2 * CHUNK)
R_TAB = 100352
SLAB = R_TAB // NS
TRASH = R_TAB - 8

_mesh = plsc.VectorSubcoreMesh(
    core_axis_name="c", subcore_axis_name="s", num_cores=NC, num_subcores=NS)


@functools.partial(
    pl.kernel,
    out_type=(
        jax.ShapeDtypeStruct((NC, R_TAB, DIM), jnp.float32),
        jax.ShapeDtypeStruct((NC, R_TAB), jnp.float32),
        jax.ShapeDtypeStruct((NUM_DAGS, 8), jnp.float32),
    ),
    mesh=_mesh,
    compiler_params=pltpu.CompilerParams(use_tc_tiling_on_sc=False,
                                         needs_layout_passes=False),
    scratch_types=[
        pltpu.VMEM_SHARED((R_TAB, DIM), jnp.float32),
        pltpu.VMEM_SHARED((R_TAB,), jnp.float32),
        pltpu.VMEM((CHUNK,), jnp.int32),
        pltpu.VMEM((CHUNK,), jnp.int32),
        pltpu.VMEM((CHUNK,), jnp.float32),
        pltpu.VMEM((CHUNK,), jnp.int32),
        pltpu.VMEM((CHUNK,), jnp.int32),
        pltpu.VMEM((CHUNK,), jnp.float32),
        pltpu.VMEM((CHUNK + SUB,), jnp.int32),
        pltpu.VMEM((CHUNK + SUB,), jnp.int32),
        pltpu.VMEM((MAXSUB, SUB), jnp.int32),
        pltpu.VMEM((MAXSUB, SUB), jnp.int32),
        pltpu.VMEM((CHUNK, DIM), jnp.float32),
        pltpu.VMEM((CHUNK, DIM), jnp.float32),
        pltpu.VMEM((SUB,), jnp.float32),
        pltpu.VMEM((NUM_DAGS,), jnp.int32),
        pltpu.VMEM((NUM_DAGS, 8), jnp.float32),
        pltpu.SemaphoreType.DMA,
        pltpu.SemaphoreType.DMA,
        pltpu.SemaphoreType.DMA,
        pltpu.SemaphoreType.DMA,
    ],
)
def _edge_sc(y_hbm, src_hbm, dst_hbm, mask_hbm, z8_hbm, z1_hbm, xp_hbm, ptr_hbm,
             agg_out, cnt_out, dagx_out,
             agg_sh, cnt_sh, src0, dst0, mk0, src1, dst1, mk1, psrc, pdst,
             sidx0, sidx1, rows0, rows1, ones_v, ptr_v, dagx_v,
             isem, gsem, ssem, csem):
    c = lax.axis_index("c")
    s = lax.axis_index("s")

    pltpu.sync_copy(z8_hbm, agg_sh.at[pl.ds(s * SLAB, SLAB)])
    pltpu.sync_copy(z1_hbm, cnt_sh.at[pl.ds(s * SLAB, SLAB)])
    for i in range(SUB // 16):
        ones_v[pl.ds(i * 16, 16)] = jnp.ones((16,), jnp.float32)
    plsc.subcore_barrier()

    base = (c * NS + s) * EPW
    trash = 100008 + s * 8

    def fire_in(ch, sv, dv, mv):
        off = base + ch * CHUNK
        pltpu.async_copy(src_hbm.at[pl.ds(off, CHUNK)], sv, isem)
        pltpu.async_copy(dst_hbm.at[pl.ds(off, CHUNK)], dv, isem)
        pltpu.async_copy(mask_hbm.at[pl.ds(off, CHUNK)], mv, isem)

    def drain_in(ch, sv, dv, mv):
        off = base + ch * CHUNK
        pltpu.make_async_copy(src_hbm.at[pl.ds(off, CHUNK)], sv, isem).wait()
        pltpu.make_async_copy(dst_hbm.at[pl.ds(off, CHUNK)], dv, isem).wait()
        pltpu.make_async_copy(mask_hbm.at[pl.ds(off, CHUNK)], mv, isem).wait()

    def compress(sv, dv, mv, sidx):
        def comp_i(i, off):
            sl = pl.ds(i * 16, 16)
            m = mv[sl] > 0.0
            mi = m.astype(jnp.int32)
            pos = plsc.cumsum(mi)
            idx = off + pos - mi
            plsc.store_scatter(psrc, [idx], sv[sl], mask=m)
            plsc.store_scatter(pdst, [idx], dv[sl], mask=m)
            return off + pos[15]
        nc_ = lax.fori_loop(0, NVEC, comp_i, 0)
        for j in range(SUB // 16):
            psrc[pl.ds(nc_ + j * 16, 16)] = jnp.zeros((16,), jnp.int32) + trash
            pdst[pl.ds(nc_ + j * 16, 16)] = jnp.zeros((16,), jnp.int32)
        nsub = (nc_ + (SUB - 1)) // SUB
        vpr = SUB // 16
        def cp(k, _):
            sidx[k // vpr, pl.ds((k % vpr) * 16, 16)] = psrc[pl.ds(k * 16, 16)]
            return 0
        lax.fori_loop(0, nsub * vpr, cp, 0)
        return nsub

    def fire_gather(nsub, rows):
        for j in range(MAXSUB):
            @pl.when(j < nsub)
            def _():
                pltpu.async_copy(y_hbm.at[pdst.at[pl.ds(j * SUB, SUB)]],
                                 rows.at[pl.ds(j * SUB, SUB)], gsem)

    def drain_gather(nsub, rows):
        for j in range(MAXSUB):
            @pl.when(j < nsub)
            def _():
                pltpu.make_async_copy(y_hbm.at[pdst.at[pl.ds(j * SUB, SUB)]],
                                      rows.at[pl.ds(j * SUB, SUB)], gsem).wait()

    def fire_scatter(nsub, rows, sidx):
        for j in range(MAXSUB):
            @pl.when(j < nsub)
            def _():
                pltpu.async_copy(rows.at[pl.ds(j * SUB, SUB)],
                                 agg_sh.at[sidx.at[j]], ssem, add=True)
                pltpu.async_copy(ones_v, cnt_sh.at[sidx.at[j]], csem, add=True)

    def drain_scatter(nsub, rows, sidx):
        for j in range(MAXSUB):
            @pl.when(j < nsub)
            def _():
                pltpu.make_async_copy(rows.at[pl.ds(j * SUB, SUB)],
                                      agg_sh.at[sidx.at[j]], ssem).wait()
                pltpu.make_async_copy(ones_v, cnt_sh.at[sidx.at[j]], csem).wait()

    fire_in(0, src0, dst0, mk0)
    fire_in(1, src1, dst1, mk1)

    def pair_body(t, ns_prev):
        drain_in(2 * t, src0, dst0, mk0)
        ns_a = compress(src0, dst0, mk0, sidx0)
        fire_in(2 * t + 2, src0, dst0, mk0)
        fire_gather(ns_a, rows0)
        drain_scatter(ns_prev, rows1, sidx1)
        drain_gather(ns_a, rows0)
        fire_scatter(ns_a, rows0, sidx0)
        drain_in(2 * t + 1, src1, dst1, mk1)
        ns_b = compress(src1, dst1, mk1, sidx1)
        fire_in(2 * t + 3, src1, dst1, mk1)
        fire_gather(ns_b, rows1)
        drain_scatter(ns_a, rows0, sidx0)
        drain_gather(ns_b, rows1)
        fire_scatter(ns_b, rows1, sidx1)
        return ns_b

    ns_last = lax.fori_loop(0, NPAIR, pair_body, 0)
    drain_scatter(ns_last, rows1, sidx1)
    drain_in(NPAIR * 2, src0, dst0, mk0)
    drain_in(NPAIR * 2 + 1, src1, dst1, mk1)

    plsc.subcore_barrier()
    pltpu.sync_copy(agg_sh.at[pl.ds(s * SLAB, SLAB)],
                    agg_out.at[c, pl.ds(s * SLAB, SLAB)])
    pltpu.sync_copy(cnt_sh.at[pl.ds(s * SLAB, SLAB)],
                    cnt_out.at[c, pl.ds(s * SLAB, SLAB)])

    @pl.when((c == 0) & (s == 0))
    def _():
        pltpu.sync_copy(ptr_hbm.at[pl.ds(0, NUM_DAGS)], ptr_v)
        pltpu.async_copy(xp_hbm.at[ptr_v], dagx_v, gsem).wait()
        pltpu.sync_copy(dagx_v, dagx_out)


_B = 2000
_G = N // _B


def _leaky(v):
    return jnp.maximum(v, 0.2 * v)


def _mm(a, w):
    return lax.dot_general(a, w, (((1,), (0,)), ((), ())),
                           preferred_element_type=jnp.float32)


def _mlp3(w, a):
    a = _leaky(_mm(a, w[0]) + w[1])
    a = _leaky(_mm(a, w[2]) + w[3])
    return _mm(a, w[4]) + w[5]


def _rows(m):
    return pl.BlockSpec((_B, m), lambda i: (i, 0))


def _full(shape):
    return pl.BlockSpec(shape, lambda i: (0,) * len(shape))


def _prep_body(x_ref, *refs):
    w = [r[...] for r in refs[:12]]
    h_ref, y_ref = refs[12:]
    h = _mlp3(w[:6], x_ref[...])
    h_ref[...] = h
    y_ref[...] = _mlp3(w[6:], h)


def _upd_body(aggA, aggB, cntA, cntB, h_ref, *refs):
    w = [r[...] for r in refs[:12]]
    hn_ref, yn_ref = refs[12:]
    agg = aggA[0] + aggB[0]
    cnt = cntA[0] + cntB[0]
    u = _mlp3(w[:6], agg)
    hn = h_ref[...] + jnp.where(cnt > 0.0, u, 0.0)
    hn_ref[...] = hn
    yn_ref[...] = _mlp3(w[6:], hn)


def _seg_onehot(plo_ref, phi_ref):
    i = pl.program_id(0)
    rowid = i * _B + lax.broadcasted_iota(jnp.int32, (_B, 1), 0)
    return ((rowid >= plo_ref[...]) & (rowid < phi_ref[...])).astype(jnp.float32)


def _dagsum_body(x_ref, h_ref, plo_ref, phi_ref, *refs):
    w = [r[...] for r in refs[:13]]
    dag_ref, glob_ref = refs[13:]
    i = pl.program_id(0)
    oh = _seg_onehot(plo_ref, phi_ref)
    z = _leaky(_mm(x_ref[...], w[0]) + _mm(h_ref[...], w[1]) + w[2])
    z = _leaky(_mm(z, w[3]) + w[4])
    z = _mm(z, w[5]) + w[6]
    part = lax.dot_general(oh, z, (((0,), (0,)), ((), ())),
                           preferred_element_type=jnp.float32)

    @pl.when(i == 0)
    def _():
        dag_ref[...] = part

    @pl.when(i > 0)
    def _():
        dag_ref[...] = dag_ref[...] + part

    @pl.when(i == _G - 1)
    def _():
        g = _mlp3(w[7:13], dag_ref[...])
        glob_ref[...] = jnp.sum(g, axis=0, keepdims=True)


def _nscore_body(x_ref, h_ref, plo_ref, phi_ref, dsum_ref, glob_ref, *refs):
    w = [r[...] for r in refs[:11]]
    ns_ref = refs[11]
    oh = _seg_onehot(plo_ref, phi_ref)
    drep = lax.dot_general(oh, dsum_ref[...], (((1,), (0,)), ((), ())),
                           preferred_element_type=jnp.float32)
    pre = (_mm(x_ref[...], w[0]) + _mm(h_ref[...], w[1]) + _mm(drep, w[2])
           + _mm(glob_ref[...], w[3]) + w[4])
    a = _leaky(pre)
    a = _leaky(_mm(a, w[5]) + w[6])
    a = _leaky(_mm(a, w[7]) + w[8])
    ns_ref[...] = _mm(a, w[9]) + w[10]


def _dscore_body(dagf_ref, dsum_ref, glob_ref, ea_ref, *refs):
    w = [r[...] for r in refs[:11]]
    out_ref = refs[11]
    m = (_mm(dagf_ref[...], w[0]) + _mm(dsum_ref[...], w[1])
         + _mm(glob_ref[...], w[2]) + w[4])
    e = ea_ref[...] * w[3]
    rows = NUM_DAGS * NEXEC
    q = ((lax.broadcasted_iota(jnp.int32, (rows, NUM_DAGS), 0) // NEXEC)
         == lax.broadcasted_iota(jnp.int32, (rows, NUM_DAGS), 1)).astype(jnp.float32)
    p = ((lax.broadcasted_iota(jnp.int32, (rows, NEXEC), 0) % NEXEC)
         == lax.broadcasted_iota(jnp.int32, (rows, NEXEC), 1)).astype(jnp.float32)
    a = _leaky(_mm(q, m) + _mm(p, e))
    a = _leaky(_mm(a, w[5]) + w[6])
    a = _leaky(_mm(a, w[7]) + w[8])
    out_ref[...] = _mm(a, w[9]) + w[10]


def _flat(layers):
    return [a for W, b in layers for a in (W, b.reshape(1, -1))]


def kernel(x, edge_index, edge_mask_batch, ptr, params):
    f32 = jnp.float32
    epad = E_PAD + CHUNK - E
    src_p = jnp.pad(edge_index[0], (0, epad)).astype(jnp.int32)
    dst_p = jnp.pad(edge_index[1], (0, epad)).astype(jnp.int32)
    maskf = jnp.pad(edge_mask_batch, ((0, 0), (0, epad))).astype(f32)
    xp8 = jnp.pad(x, ((0, 0), (0, 8 - x.shape[1])))
    ptr_i = ptr.astype(jnp.int32)
    z8 = jnp.zeros((SLAB, DIM), f32)
    z1 = jnp.zeros((SLAB,), f32)
    plo = ptr_i[:-1].reshape(1, NUM_DAGS)
    phi = ptr_i[1:].reshape(1, NUM_DAGS)
    ea = (jnp.arange(NEXEC, dtype=f32) / NEXEC).reshape(NEXEC, 1)

    w_prep = _flat(params['node_prep'])
    w_msg = _flat(params['node_msg'])
    w_upd = _flat(params['node_update'])
    w_dagm = _flat(params['dag_msg'])
    w_glob = _flat(params['glob_msg'])
    w_ns = _flat(params['node_score'])
    w_ds = _flat(params['dag_score'])

    wspecs = lambda ws: [_full(w.shape) for w in ws]

    h, y = pl.pallas_call(
        _prep_body,
        grid=(_G,),
        in_specs=[_rows(5)] + wspecs(w_prep + w_msg),
        out_specs=[_rows(DIM), _rows(DIM)],
        out_shape=[jax.ShapeDtypeStruct((N, DIM), f32)] * 2,
    )(x, *w_prep, *w_msg)

    dagx = None
    for d in range(DEPTH):
        agg2, cnt2, dagx = _edge_sc(
            y, src_p, dst_p, maskf[d], z8, z1, xp8, ptr_i)
        cnt3 = cnt2.reshape(NC, R_TAB, 1)
        h, y = pl.pallas_call(
            _upd_body,
            grid=(_G,),
            in_specs=[
                pl.BlockSpec((1, _B, DIM), lambda i: (0, i, 0)),
                pl.BlockSpec((1, _B, DIM), lambda i: (1, i, 0)),
                pl.BlockSpec((1, _B, 1), lambda i: (0, i, 0)),
                pl.BlockSpec((1, _B, 1), lambda i: (1, i, 0)),
                _rows(DIM),
            ] + wspecs(w_upd + w_msg),
            out_specs=[_rows(DIM), _rows(DIM)],
            out_shape=[jax.ShapeDtypeStruct((N, DIM), f32)] * 2,
        )(agg2, agg2, cnt3, cnt3, h, *w_upd, *w_msg)

    w1 = params['dag_msg'][0][0]
    w_dag_split = ([w1[:5], w1[5:], w_dagm[1]] + w_dagm[2:])
    dsum, glob = pl.pallas_call(
        _dagsum_body,
        grid=(_G,),
        in_specs=[_rows(5), _rows(DIM), _full((1, NUM_DAGS)), _full((1, NUM_DAGS))]
        + wspecs(w_dag_split + w_glob),
        out_specs=[_full((NUM_DAGS, DIM)), _full((1, DIM))],
        out_shape=[jax.ShapeDtypeStruct((NUM_DAGS, DIM), f32),
                   jax.ShapeDtypeStruct((1, DIM), f32)],
    )(x, h, plo, phi, *w_dag_split, *w_glob)

    wn1 = params['node_score'][0][0]
    w_ns_split = ([wn1[:5], wn1[5:13], wn1[13:21], wn1[21:29], w_ns[1]]
                  + w_ns[2:])
    ns = pl.pallas_call(
        _nscore_body,
        grid=(_G,),
        in_specs=[_rows(5), _rows(DIM), _full((1, NUM_DAGS)), _full((1, NUM_DAGS)),
                  _full((NUM_DAGS, DIM)), _full((1, DIM))] + wspecs(w_ns_split),
        out_specs=[_rows(1)],
        out_shape=[jax.ShapeDtypeStruct((N, 1), f32)],
    )(x, h, plo, phi, dsum, glob, *w_ns_split)[0]

    wd1 = params['dag_score'][0][0]
    w_ds_split = ([wd1[:3], wd1[3:11], wd1[11:19], wd1[19:20].reshape(1, -1),
                   w_ds[1]] + w_ds[2:])
    dagf = dagx[:, :3]
    dsc = pl.pallas_call(
        _dscore_body,
        grid=(1,),
        in_specs=[_full((NUM_DAGS, 3)), _full((NUM_DAGS, DIM)), _full((1, DIM)),
                  _full((NEXEC, 1))] + wspecs(w_ds_split),
        out_specs=[_full((NUM_DAGS * NEXEC, 1))],
        out_shape=[jax.ShapeDtypeStruct((NUM_DAGS * NEXEC, 1), f32)],
    )(dagf, dsum, glob, ea, *w_ds_split)[0]

    return ns.reshape(N), dsc.reshape(NUM_DAGS, NEXEC)


# speedup vs baseline: 3.4418x; 3.4394x over previous
import functools

import jax
import jax.numpy as jnp
from jax import lax
from jax.experimental import pallas as pl
from jax.experimental.pallas import tpu as pltpu
from jax.experimental.pallas import tpu_sc as plsc

N = 100000
E = 3200000
DIM = 8
DEPTH = 8
NUM_DAGS = 128
NEXEC = 50

NC = 2
NS = 16
NW = NC * NS
EPW = 102400
E_PAD = NW * EPW
CHUNK = 1024
NVEC = CHUNK // 16
SUB = 256
MAXSUB = CHUNK // SUB
NPAIR = EPW // (2 * CHUNK)
R_TAB = 100352
SLAB = R_TAB // NS
TRASH = R_TAB - 8

_mesh = plsc.VectorSubcoreMesh(
    core_axis_name="c", subcore_axis_name="s", num_cores=NC, num_subcores=NS)


@functools.partial(
    pl.kernel,
    out_type=(
        jax.ShapeDtypeStruct((NC, R_TAB, DIM), jnp.float32),
        jax.ShapeDtypeStruct((NC, R_TAB), jnp.float32),
        jax.ShapeDtypeStruct((NUM_DAGS, 8), jnp.float32),
    ),
    mesh=_mesh,
    compiler_params=pltpu.CompilerParams(use_tc_tiling_on_sc=False,
                                         needs_layout_passes=False),
    scratch_types=[
        pltpu.VMEM_SHARED((R_TAB, DIM), jnp.float32),
        pltpu.VMEM_SHARED((R_TAB,), jnp.float32),
        pltpu.VMEM((CHUNK,), jnp.int32),
        pltpu.VMEM((CHUNK,), jnp.int32),
        pltpu.VMEM((CHUNK,), jnp.float32),
        pltpu.VMEM((CHUNK,), jnp.int32),
        pltpu.VMEM((CHUNK,), jnp.int32),
        pltpu.VMEM((CHUNK,), jnp.float32),
        pltpu.VMEM((CHUNK + SUB,), jnp.int32),
        pltpu.VMEM((CHUNK + SUB,), jnp.int32),
        pltpu.VMEM((MAXSUB, SUB), jnp.int32),
        pltpu.VMEM((MAXSUB, SUB), jnp.int32),
        pltpu.VMEM((CHUNK, DIM), jnp.float32),
        pltpu.VMEM((CHUNK, DIM), jnp.float32),
        pltpu.VMEM((SUB,), jnp.float32),
        pltpu.VMEM((NUM_DAGS,), jnp.int32),
        pltpu.VMEM((NUM_DAGS, 8), jnp.float32),
        pltpu.SemaphoreType.DMA,
        pltpu.SemaphoreType.DMA,
        pltpu.SemaphoreType.DMA,
        pltpu.SemaphoreType.DMA,
    ],
)
def _edge_sc(y_hbm, src_hbm, dst_hbm, mask_hbm, z8_hbm, z1_hbm, xp_hbm, ptr_hbm,
             agg_out, cnt_out, dagx_out,
             agg_sh, cnt_sh, src0, dst0, mk0, src1, dst1, mk1, psrc, pdst,
             sidx0, sidx1, rows0, rows1, ones_v, ptr_v, dagx_v,
             isem, gsem, ssem, csem):
    c = lax.axis_index("c")
    s = lax.axis_index("s")

    pltpu.sync_copy(z8_hbm, agg_sh.at[pl.ds(s * SLAB, SLAB)])
    pltpu.sync_copy(z1_hbm, cnt_sh.at[pl.ds(s * SLAB, SLAB)])
    for i in range(SUB // 16):
        ones_v[pl.ds(i * 16, 16)] = jnp.ones((16,), jnp.float32)
    plsc.subcore_barrier()

    base = (c * NS + s) * EPW
    trash = 100008 + s * 8

    def fire_in(ch, sv, dv, mv):
        off = base + ch * CHUNK
        pltpu.async_copy(src_hbm.at[pl.ds(off, CHUNK)], sv, isem)
        pltpu.async_copy(dst_hbm.at[pl.ds(off, CHUNK)], dv, isem)
        pltpu.async_copy(mask_hbm.at[pl.ds(off, CHUNK)], mv, isem)

    def drain_in(ch, sv, dv, mv):
        off = base + ch * CHUNK
        pltpu.make_async_copy(src_hbm.at[pl.ds(off, CHUNK)], sv, isem).wait()
        pltpu.make_async_copy(dst_hbm.at[pl.ds(off, CHUNK)], dv, isem).wait()
        pltpu.make_async_copy(mask_hbm.at[pl.ds(off, CHUNK)], mv, isem).wait()

    def compress(sv, dv, mv, sidx):
        def comp_i(i, off):
            sl = pl.ds(i * 16, 16)
            m = mv[sl] > 0.0
            mi = m.astype(jnp.int32)
            pos = plsc.cumsum(mi)
            idx = off + pos - mi
            plsc.store_scatter(psrc, [idx], sv[sl], mask=m)
            plsc.store_scatter(pdst, [idx], dv[sl], mask=m)
            return off + pos[15]
        nc_ = lax.fori_loop(0, NVEC, comp_i, 0)
        for j in range(SUB // 16):
            psrc[pl.ds(nc_ + j * 16, 16)] = jnp.zeros((16,), jnp.int32) + trash
            pdst[pl.ds(nc_ + j * 16, 16)] = jnp.zeros((16,), jnp.int32)
        nsub = (nc_ + (SUB - 1)) // SUB
        vpr = SUB // 16
        def cp(k, _):
            sidx[k // vpr, pl.ds((k % vpr) * 16, 16)] = psrc[pl.ds(k * 16, 16)]
            return 0
        lax.fori_loop(0, nsub * vpr, cp, 0)
        return nsub

    def fire_gather(nsub, rows):
        for j in range(MAXSUB):
            @pl.when(j < nsub)
            def _():
                pltpu.async_copy(y_hbm.at[pdst.at[pl.ds(j * SUB, SUB)]],
                                 rows.at[pl.ds(j * SUB, SUB)], gsem)

    def drain_gather(nsub, rows):
        for j in range(MAXSUB):
            @pl.when(j < nsub)
            def _():
                pltpu.make_async_copy(y_hbm.at[pdst.at[pl.ds(j * SUB, SUB)]],
                                      rows.at[pl.ds(j * SUB, SUB)], gsem).wait()

    def fire_scatter(nsub, rows, sidx):
        for j in range(MAXSUB):
            @pl.when(j < nsub)
            def _():
                pltpu.async_copy(rows.at[pl.ds(j * SUB, SUB)],
                                 agg_sh.at[sidx.at[j]], ssem, add=True)
                pltpu.async_copy(ones_v, cnt_sh.at[sidx.at[j]], csem, add=True)

    def drain_scatter(nsub, rows, sidx):
        for j in range(MAXSUB):
            @pl.when(j < nsub)
            def _():
                pltpu.make_async_copy(rows.at[pl.ds(j * SUB, SUB)],
                                      agg_sh.at[sidx.at[j]], ssem).wait()
                pltpu.make_async_copy(ones_v, cnt_sh.at[sidx.at[j]], csem).wait()

    fire_in(0, src0, dst0, mk0)
    fire_in(1, src1, dst1, mk1)

    def pair_body(t, ns_prev):
        drain_in(2 * t, src0, dst0, mk0)
        ns_a = compress(src0, dst0, mk0, sidx0)
        fire_in(2 * t + 2, src0, dst0, mk0)
        fire_gather(ns_a, rows0)
        drain_scatter(ns_prev, rows1, sidx1)
        drain_gather(ns_a, rows0)
        fire_scatter(ns_a, rows0, sidx0)
        drain_in(2 * t + 1, src1, dst1, mk1)
        ns_b = compress(src1, dst1, mk1, sidx1)
        fire_in(2 * t + 3, src1, dst1, mk1)
        fire_gather(ns_b, rows1)
        drain_scatter(ns_a, rows0, sidx0)
        drain_gather(ns_b, rows1)
        fire_scatter(ns_b, rows1, sidx1)
        return ns_b

    ns_last = lax.fori_loop(0, NPAIR, pair_body, 0)
    drain_scatter(ns_last, rows1, sidx1)
    drain_in(NPAIR * 2, src0, dst0, mk0)
    drain_in(NPAIR * 2 + 1, src1, dst1, mk1)

    plsc.subcore_barrier()
    pltpu.sync_copy(agg_sh.at[pl.ds(s * SLAB, SLAB)],
                    agg_out.at[c, pl.ds(s * SLAB, SLAB)])
    pltpu.sync_copy(cnt_sh.at[pl.ds(s * SLAB, SLAB)],
                    cnt_out.at[c, pl.ds(s * SLAB, SLAB)])

    @pl.when((c == 0) & (s == 0))
    def _():
        pltpu.sync_copy(ptr_hbm.at[pl.ds(0, NUM_DAGS)], ptr_v)
        pltpu.async_copy(xp_hbm.at[ptr_v], dagx_v, gsem).wait()
        pltpu.sync_copy(dagx_v, dagx_out)


_B = 2000
_G = N // _B


def _leaky(v):
    return jnp.maximum(v, 0.2 * v)


def _mm(a, w):
    return lax.dot_general(a, w, (((1,), (0,)), ((), ())),
                           preferred_element_type=jnp.float32)


def _mlp3(w, a):
    a = _leaky(_mm(a, w[0]) + w[1])
    a = _leaky(_mm(a, w[2]) + w[3])
    return _mm(a, w[4]) + w[5]


def _rows(m):
    return pl.BlockSpec((_B, m), lambda i: (i, 0))


def _full(shape):
    return pl.BlockSpec(shape, lambda i: (0,) * len(shape))


def _prep_body(x_ref, *refs):
    w = [r[...] for r in refs[:12]]
    h_ref, y_ref = refs[12:]
    h = _mlp3(w[:6], x_ref[...])
    h_ref[...] = h
    y_ref[...] = _mlp3(w[6:], h)


def _upd_body(aggA, aggB, cntA, cntB, h_ref, *refs):
    w = [r[...] for r in refs[:12]]
    hn_ref, yn_ref = refs[12:]
    agg = aggA[0] + aggB[0]
    cnt = cntA[0] + cntB[0]
    u = _mlp3(w[:6], agg)
    hn = h_ref[...] + jnp.where(cnt > 0.0, u, 0.0)
    hn_ref[...] = hn
    yn_ref[...] = _mlp3(w[6:], hn)


def _seg_onehot(plo_ref, phi_ref):
    i = pl.program_id(0)
    rowid = i * _B + lax.broadcasted_iota(jnp.int32, (_B, 1), 0)
    return ((rowid >= plo_ref[...]) & (rowid < phi_ref[...])).astype(jnp.float32)


def _dagsum_body(x_ref, h_ref, plo_ref, phi_ref, *refs):
    w = [r[...] for r in refs[:13]]
    dag_ref, glob_ref = refs[13:]
    i = pl.program_id(0)
    oh = _seg_onehot(plo_ref, phi_ref)
    z = _leaky(_mm(x_ref[...], w[0]) + _mm(h_ref[...], w[1]) + w[2])
    z = _leaky(_mm(z, w[3]) + w[4])
    z = _mm(z, w[5]) + w[6]
    part = lax.dot_general(oh, z, (((0,), (0,)), ((), ())),
                           preferred_element_type=jnp.float32)

    @pl.when(i == 0)
    def _():
        dag_ref[...] = part

    @pl.when(i > 0)
    def _():
        dag_ref[...] = dag_ref[...] + part

    @pl.when(i == _G - 1)
    def _():
        g = _mlp3(w[7:13], dag_ref[...])
        glob_ref[...] = jnp.sum(g, axis=0, keepdims=True)


def _nscore_body(x_ref, h_ref, plo_ref, phi_ref, dsum_ref, glob_ref, *refs):
    w = [r[...] for r in refs[:11]]
    ns_ref = refs[11]
    oh = _seg_onehot(plo_ref, phi_ref)
    drep = lax.dot_general(oh, dsum_ref[...], (((1,), (0,)), ((), ())),
                           preferred_element_type=jnp.float32)
    pre = (_mm(x_ref[...], w[0]) + _mm(h_ref[...], w[1]) + _mm(drep, w[2])
           + _mm(glob_ref[...], w[3]) + w[4])
    a = _leaky(pre)
    a = _leaky(_mm(a, w[5]) + w[6])
    a = _leaky(_mm(a, w[7]) + w[8])
    ns_ref[...] = _mm(a, w[9]) + w[10]


def _dscore_body(dagf_ref, dsum_ref, glob_ref, ea_ref, *refs):
    w = [r[...] for r in refs[:11]]
    out_ref = refs[11]
    m = (_mm(dagf_ref[...], w[0]) + _mm(dsum_ref[...], w[1])
         + _mm(glob_ref[...], w[2]) + w[4])
    e = ea_ref[...] * w[3]
    rows = NUM_DAGS * NEXEC
    q = ((lax.broadcasted_iota(jnp.int32, (rows, NUM_DAGS), 0) // NEXEC)
         == lax.broadcasted_iota(jnp.int32, (rows, NUM_DAGS), 1)).astype(jnp.float32)
    p = ((lax.broadcasted_iota(jnp.int32, (rows, NEXEC), 0) % NEXEC)
         == lax.broadcasted_iota(jnp.int32, (rows, NEXEC), 1)).astype(jnp.float32)
    a = _leaky(_mm(q, m) + _mm(p, e))
    a = _leaky(_mm(a, w[5]) + w[6])
    a = _leaky(_mm(a, w[7]) + w[8])
    out_ref[...] = _mm(a, w[9]) + w[10]


def _flat(layers):
    return [a for W, b in layers for a in (W, b.reshape(1, -1))]


def kernel(x, edge_index, edge_mask_batch, ptr, params):
    f32 = jnp.float32
    epad = E_PAD + CHUNK - E
    src_p = jnp.pad(edge_index[0], (0, epad)).astype(jnp.int32)
    dst_p = jnp.pad(edge_index[1], (0, epad)).astype(jnp.int32)
    maskf = jnp.pad(edge_mask_batch, ((0, 0), (0, epad))).astype(f32)
    xp8 = jnp.pad(x, ((0, 0), (0, 8 - x.shape[1])))
    ptr_i = ptr.astype(jnp.int32)
    z8 = jnp.zeros((SLAB, DIM), f32)
    z1 = jnp.zeros((SLAB,), f32)
    plo = ptr_i[:-1].reshape(1, NUM_DAGS)
    phi = ptr_i[1:].reshape(1, NUM_DAGS)
    ea = (jnp.arange(NEXEC, dtype=f32) / NEXEC).reshape(NEXEC, 1)

    w_prep = _flat(params['node_prep'])
    w_msg = _flat(params['node_msg'])
    w_upd = _flat(params['node_update'])
    w_dagm = _flat(params['dag_msg'])
    w_glob = _flat(params['glob_msg'])
    w_ns = _flat(params['node_score'])
    w_ds = _flat(params['dag_score'])

    wspecs = lambda ws: [_full(w.shape) for w in ws]

    h, y = pl.pallas_call(
        _prep_body,
        grid=(_G,),
        in_specs=[_rows(5)] + wspecs(w_prep + w_msg),
        out_specs=[_rows(DIM), _rows(DIM)],
        out_shape=[jax.ShapeDtypeStruct((N, DIM), f32)] * 2,
    )(x, *w_prep, *w_msg)

    dagx = None
    for d in range(DEPTH):
        agg2, cnt2, dagx = _edge_sc(
            y, src_p, dst_p, maskf[d], z8, z1, xp8, ptr_i)
        cnt3 = cnt2.reshape(NC, R_TAB, 1)
        h, y = pl.pallas_call(
            _upd_body,
            grid=(_G,),
            in_specs=[
                pl.BlockSpec((1, _B, DIM), lambda i: (0, i, 0)),
                pl.BlockSpec((1, _B, DIM), lambda i: (1, i, 0)),
                pl.BlockSpec((1, _B, 1), lambda i: (0, i, 0)),
                pl.BlockSpec((1, _B, 1), lambda i: (1, i, 0)),
                _rows(DIM),
            ] + wspecs(w_upd + w_msg),
            out_specs=[_rows(DIM), _rows(DIM)],
            out_shape=[jax.ShapeDtypeStruct((N, DIM), f32)] * 2,
        )(agg2, agg2, cnt3, cnt3, h, *w_upd, *w_msg)

    w1 = params['dag_msg'][0][0]
    w_dag_split = ([w1[:5], w1[5:], w_dagm[1]] + w_dagm[2:])
    dsum, glob = pl.pallas_call(
        _dagsum_body,
        grid=(_G,),
        in_specs=[_rows(5), _rows(DIM), _full((1, NUM_DAGS)), _full((1, NUM_DAGS))]
        + wspecs(w_dag_split + w_glob),
        out_specs=[_full((NUM_DAGS, DIM)), _full((1, DIM))],
        out_shape=[jax.ShapeDtypeStruct((NUM_DAGS, DIM), f32),
                   jax.ShapeDtypeStruct((1, DIM), f32)],
    )(x, h, plo, phi, *w_dag_split, *w_glob)

    wn1 = params['node_score'][0][0]
    w_ns_split = ([wn1[:5], wn1[5:13], wn1[13:21], wn1[21:29], w_ns[1]]
                  + w_ns[2:])
    ns = pl.pallas_call(
        _nscore_body,
        grid=(_G,),
        in_specs=[_rows(5), _rows(DIM), _full((1, NUM_DAGS)), _full((1, NUM_DAGS)),
                  _full((NUM_DAGS, DIM)), _full((1, DIM))] + wspecs(w_ns_split),
        out_specs=[_rows(1)],
        out_shape=[jax.ShapeDtypeStruct((N, 1), f32)],
    )(x, h, plo, phi, dsum, glob, *w_ns_split)[0]

    wd1 = params['dag_score'][0][0]
    w_ds_split = ([wd1[:3], wd1[3:11], wd1[11:19], wd1[19:20].reshape(1, -1),
                   w_ds[1]] + w_ds[2:])
    dagf = dagx[:, :3]
    dsc = pl.pallas_call(
        _dscore_body,
        grid=(1,),
        in_specs=[_full((NUM_DAGS, 3)), _full((NUM_DAGS, DIM)), _full((1, DIM)),
                  _full((NEXEC, 1))] + wspecs(w_ds_split),
        out_specs=[_full((NUM_DAGS * NEXEC, 1))],
        out_shape=[jax.ShapeDtypeStruct((NUM_DAGS * NEXEC, 1), f32)],
    )(dagf, dsum, glob, ea, *w_ds_split)[0]

    return ns.reshape(N), dsc.reshape(NUM_DAGS, NEXEC)


# speedup vs baseline: 5.6863x; 1.6521x over previous
import functools

import jax
import jax.numpy as jnp
from jax import lax
from jax.experimental import pallas as pl
from jax.experimental.pallas import tpu as pltpu
from jax.experimental.pallas import tpu_sc as plsc

N = 100000
E = 3200000
DIM = 8
DEPTH = 8
NUM_DAGS = 128
NEXEC = 50

NC = 2
NS = 16
NW = NC * NS
EPW = 102400
E_PAD = NW * EPW
CHUNK = 1024
NVEC = CHUNK // 16
SUB = 128
MAXSUB = CHUNK // SUB
NPAIR = EPW // (2 * CHUNK)
R_TAB = 100352
SLAB = R_TAB // NS
TRASH = R_TAB - 8

_mesh = plsc.VectorSubcoreMesh(
    core_axis_name="c", subcore_axis_name="s", num_cores=NC, num_subcores=NS)


@functools.partial(
    pl.kernel,
    out_type=(
        jax.ShapeDtypeStruct((NC, R_TAB, DIM), jnp.float32),
        jax.ShapeDtypeStruct((NC, R_TAB), jnp.float32),
        jax.ShapeDtypeStruct((NUM_DAGS, 8), jnp.float32),
    ),
    mesh=_mesh,
    compiler_params=pltpu.CompilerParams(use_tc_tiling_on_sc=False,
                                         needs_layout_passes=False),
    scratch_types=[
        pltpu.VMEM_SHARED((R_TAB, DIM), jnp.float32),
        pltpu.VMEM_SHARED((R_TAB,), jnp.float32),
        pltpu.VMEM((CHUNK,), jnp.int32),
        pltpu.VMEM((CHUNK,), jnp.int32),
        pltpu.VMEM((CHUNK,), jnp.float32),
        pltpu.VMEM((CHUNK,), jnp.int32),
        pltpu.VMEM((CHUNK,), jnp.int32),
        pltpu.VMEM((CHUNK,), jnp.float32),
        pltpu.VMEM((CHUNK + SUB,), jnp.int32),
        pltpu.VMEM((CHUNK + SUB,), jnp.int32),
        pltpu.VMEM((MAXSUB, SUB), jnp.int32),
        pltpu.VMEM((MAXSUB, SUB), jnp.int32),
        pltpu.VMEM((CHUNK, DIM), jnp.float32),
        pltpu.VMEM((CHUNK, DIM), jnp.float32),
        pltpu.VMEM((SUB,), jnp.float32),
        pltpu.VMEM((NUM_DAGS,), jnp.int32),
        pltpu.VMEM((NUM_DAGS, 8), jnp.float32),
        pltpu.SemaphoreType.DMA,
        pltpu.SemaphoreType.DMA,
        pltpu.SemaphoreType.DMA,
        pltpu.SemaphoreType.DMA,
    ],
)
def _edge_sc(y_hbm, src_hbm, dst_hbm, mask_hbm, z8_hbm, z1_hbm, xp_hbm, ptr_hbm,
             agg_out, cnt_out, dagx_out,
             agg_sh, cnt_sh, src0, dst0, mk0, src1, dst1, mk1, psrc, pdst,
             sidx0, sidx1, rows0, rows1, ones_v, ptr_v, dagx_v,
             isem, gsem, ssem, csem):
    c = lax.axis_index("c")
    s = lax.axis_index("s")

    pltpu.sync_copy(z8_hbm, agg_sh.at[pl.ds(s * SLAB, SLAB)])
    pltpu.sync_copy(z1_hbm, cnt_sh.at[pl.ds(s * SLAB, SLAB)])
    for i in range(SUB // 16):
        ones_v[pl.ds(i * 16, 16)] = jnp.ones((16,), jnp.float32)
    plsc.subcore_barrier()

    base = (c * NS + s) * EPW
    trash = 100008 + s * 8

    def fire_in(ch, sv, dv, mv):
        off = base + ch * CHUNK
        pltpu.async_copy(src_hbm.at[pl.ds(off, CHUNK)], sv, isem)
        pltpu.async_copy(dst_hbm.at[pl.ds(off, CHUNK)], dv, isem)
        pltpu.async_copy(mask_hbm.at[pl.ds(off, CHUNK)], mv, isem)

    def drain_in(ch, sv, dv, mv):
        off = base + ch * CHUNK
        pltpu.make_async_copy(src_hbm.at[pl.ds(off, CHUNK)], sv, isem).wait()
        pltpu.make_async_copy(dst_hbm.at[pl.ds(off, CHUNK)], dv, isem).wait()
        pltpu.make_async_copy(mask_hbm.at[pl.ds(off, CHUNK)], mv, isem).wait()

    def compress(sv, dv, mv, sidx):
        def comp_i(i, off):
            sl = pl.ds(i * 16, 16)
            m = mv[sl] > 0.0
            mi = m.astype(jnp.int32)
            pos = plsc.cumsum(mi)
            idx = off + pos - mi
            plsc.store_scatter(psrc, [idx], sv[sl], mask=m)
            plsc.store_scatter(pdst, [idx], dv[sl], mask=m)
            return off + pos[15]
        nc_ = lax.fori_loop(0, NVEC, comp_i, 0)
        for j in range(SUB // 16):
            psrc[pl.ds(nc_ + j * 16, 16)] = jnp.zeros((16,), jnp.int32) + trash
            pdst[pl.ds(nc_ + j * 16, 16)] = jnp.zeros((16,), jnp.int32)
        nsub = (nc_ + (SUB - 1)) // SUB
        vpr = SUB // 16
        def cp(k, _):
            sidx[k // vpr, pl.ds((k % vpr) * 16, 16)] = psrc[pl.ds(k * 16, 16)]
            return 0
        lax.fori_loop(0, nsub * vpr, cp, 0)
        return nsub

    def fire_gather(nsub, rows):
        for j in range(MAXSUB):
            @pl.when(j < nsub)
            def _():
                pltpu.async_copy(y_hbm.at[pdst.at[pl.ds(j * SUB, SUB)]],
                                 rows.at[pl.ds(j * SUB, SUB)], gsem)

    def drain_gather(nsub, rows):
        for j in range(MAXSUB):
            @pl.when(j < nsub)
            def _():
                pltpu.make_async_copy(y_hbm.at[pdst.at[pl.ds(j * SUB, SUB)]],
                                      rows.at[pl.ds(j * SUB, SUB)], gsem).wait()

    def fire_scatter(nsub, rows, sidx):
        for j in range(MAXSUB):
            @pl.when(j < nsub)
            def _():
                pltpu.async_copy(rows.at[pl.ds(j * SUB, SUB)],
                                 agg_sh.at[sidx.at[j]], ssem, add=True)
                pltpu.async_copy(ones_v, cnt_sh.at[sidx.at[j]], csem, add=True)

    def drain_scatter(nsub, rows, sidx):
        for j in range(MAXSUB):
            @pl.when(j < nsub)
            def _():
                pltpu.make_async_copy(rows.at[pl.ds(j * SUB, SUB)],
                                      agg_sh.at[sidx.at[j]], ssem).wait()
                pltpu.make_async_copy(ones_v, cnt_sh.at[sidx.at[j]], csem).wait()

    fire_in(0, src0, dst0, mk0)
    fire_in(1, src1, dst1, mk1)

    def pair_body(t, ns_prev):
        drain_in(2 * t, src0, dst0, mk0)
        ns_a = compress(src0, dst0, mk0, sidx0)
        fire_in(2 * t + 2, src0, dst0, mk0)
        fire_gather(ns_a, rows0)
        drain_scatter(ns_prev, rows1, sidx1)
        drain_gather(ns_a, rows0)
        fire_scatter(ns_a, rows0, sidx0)
        drain_in(2 * t + 1, src1, dst1, mk1)
        ns_b = compress(src1, dst1, mk1, sidx1)
        fire_in(2 * t + 3, src1, dst1, mk1)
        fire_gather(ns_b, rows1)
        drain_scatter(ns_a, rows0, sidx0)
        drain_gather(ns_b, rows1)
        fire_scatter(ns_b, rows1, sidx1)
        return ns_b

    ns_last = lax.fori_loop(0, NPAIR, pair_body, 0)
    drain_scatter(ns_last, rows1, sidx1)
    drain_in(NPAIR * 2, src0, dst0, mk0)
    drain_in(NPAIR * 2 + 1, src1, dst1, mk1)

    plsc.subcore_barrier()
    pltpu.sync_copy(agg_sh.at[pl.ds(s * SLAB, SLAB)],
                    agg_out.at[c, pl.ds(s * SLAB, SLAB)])
    pltpu.sync_copy(cnt_sh.at[pl.ds(s * SLAB, SLAB)],
                    cnt_out.at[c, pl.ds(s * SLAB, SLAB)])

    @pl.when((c == 0) & (s == 0))
    def _():
        pltpu.sync_copy(ptr_hbm.at[pl.ds(0, NUM_DAGS)], ptr_v)
        pltpu.async_copy(xp_hbm.at[ptr_v], dagx_v, gsem).wait()
        pltpu.sync_copy(dagx_v, dagx_out)


_B = 2000
_G = N // _B


def _leaky(v):
    return jnp.maximum(v, 0.2 * v)


def _mm(a, w):
    return lax.dot_general(a, w, (((1,), (0,)), ((), ())),
                           preferred_element_type=jnp.float32)


def _mlp3(w, a):
    a = _leaky(_mm(a, w[0]) + w[1])
    a = _leaky(_mm(a, w[2]) + w[3])
    return _mm(a, w[4]) + w[5]


def _rows(m):
    return pl.BlockSpec((_B, m), lambda i: (i, 0))


def _full(shape):
    return pl.BlockSpec(shape, lambda i: (0,) * len(shape))


def _prep_body(x_ref, *refs):
    w = [r[...] for r in refs[:12]]
    h_ref, y_ref = refs[12:]
    h = _mlp3(w[:6], x_ref[...])
    h_ref[...] = h
    y_ref[...] = _mlp3(w[6:], h)


def _upd_body(aggA, aggB, cntA, cntB, h_ref, *refs):
    w = [r[...] for r in refs[:12]]
    hn_ref, yn_ref = refs[12:]
    agg = aggA[0] + aggB[0]
    cnt = cntA[0] + cntB[0]
    u = _mlp3(w[:6], agg)
    hn = h_ref[...] + jnp.where(cnt > 0.0, u, 0.0)
    hn_ref[...] = hn
    yn_ref[...] = _mlp3(w[6:], hn)


def _seg_onehot(plo_ref, phi_ref):
    i = pl.program_id(0)
    rowid = i * _B + lax.broadcasted_iota(jnp.int32, (_B, 1), 0)
    return ((rowid >= plo_ref[...]) & (rowid < phi_ref[...])).astype(jnp.float32)


def _dagsum_body(x_ref, h_ref, plo_ref, phi_ref, *refs):
    w = [r[...] for r in refs[:13]]
    dag_ref, glob_ref = refs[13:]
    i = pl.program_id(0)
    oh = _seg_onehot(plo_ref, phi_ref)
    z = _leaky(_mm(x_ref[...], w[0]) + _mm(h_ref[...], w[1]) + w[2])
    z = _leaky(_mm(z, w[3]) + w[4])
    z = _mm(z, w[5]) + w[6]
    part = lax.dot_general(oh, z, (((0,), (0,)), ((), ())),
                           preferred_element_type=jnp.float32)

    @pl.when(i == 0)
    def _():
        dag_ref[...] = part

    @pl.when(i > 0)
    def _():
        dag_ref[...] = dag_ref[...] + part

    @pl.when(i == _G - 1)
    def _():
        g = _mlp3(w[7:13], dag_ref[...])
        glob_ref[...] = jnp.sum(g, axis=0, keepdims=True)


def _nscore_body(x_ref, h_ref, plo_ref, phi_ref, dsum_ref, glob_ref, *refs):
    w = [r[...] for r in refs[:11]]
    ns_ref = refs[11]
    oh = _seg_onehot(plo_ref, phi_ref)
    drep = lax.dot_general(oh, dsum_ref[...], (((1,), (0,)), ((), ())),
                           preferred_element_type=jnp.float32)
    pre = (_mm(x_ref[...], w[0]) + _mm(h_ref[...], w[1]) + _mm(drep, w[2])
           + _mm(glob_ref[...], w[3]) + w[4])
    a = _leaky(pre)
    a = _leaky(_mm(a, w[5]) + w[6])
    a = _leaky(_mm(a, w[7]) + w[8])
    ns_ref[...] = _mm(a, w[9]) + w[10]


def _dscore_body(dagf_ref, dsum_ref, glob_ref, ea_ref, *refs):
    w = [r[...] for r in refs[:11]]
    out_ref = refs[11]
    m = (_mm(dagf_ref[...], w[0]) + _mm(dsum_ref[...], w[1])
         + _mm(glob_ref[...], w[2]) + w[4])
    e = ea_ref[...] * w[3]
    rows = NUM_DAGS * NEXEC
    q = ((lax.broadcasted_iota(jnp.int32, (rows, NUM_DAGS), 0) // NEXEC)
         == lax.broadcasted_iota(jnp.int32, (rows, NUM_DAGS), 1)).astype(jnp.float32)
    p = ((lax.broadcasted_iota(jnp.int32, (rows, NEXEC), 0) % NEXEC)
         == lax.broadcasted_iota(jnp.int32, (rows, NEXEC), 1)).astype(jnp.float32)
    a = _leaky(_mm(q, m) + _mm(p, e))
    a = _leaky(_mm(a, w[5]) + w[6])
    a = _leaky(_mm(a, w[7]) + w[8])
    out_ref[...] = _mm(a, w[9]) + w[10]


def _flat(layers):
    return [a for W, b in layers for a in (W, b.reshape(1, -1))]


def kernel(x, edge_index, edge_mask_batch, ptr, params):
    f32 = jnp.float32
    epad = E_PAD + CHUNK - E
    src_p = jnp.pad(edge_index[0], (0, epad)).astype(jnp.int32)
    dst_p = jnp.pad(edge_index[1], (0, epad)).astype(jnp.int32)
    maskf = jnp.pad(edge_mask_batch, ((0, 0), (0, epad))).astype(f32)
    xp8 = jnp.pad(x, ((0, 0), (0, 8 - x.shape[1])))
    ptr_i = ptr.astype(jnp.int32)
    z8 = jnp.zeros((SLAB, DIM), f32)
    z1 = jnp.zeros((SLAB,), f32)
    plo = ptr_i[:-1].reshape(1, NUM_DAGS)
    phi = ptr_i[1:].reshape(1, NUM_DAGS)
    ea = (jnp.arange(NEXEC, dtype=f32) / NEXEC).reshape(NEXEC, 1)

    w_prep = _flat(params['node_prep'])
    w_msg = _flat(params['node_msg'])
    w_upd = _flat(params['node_update'])
    w_dagm = _flat(params['dag_msg'])
    w_glob = _flat(params['glob_msg'])
    w_ns = _flat(params['node_score'])
    w_ds = _flat(params['dag_score'])

    wspecs = lambda ws: [_full(w.shape) for w in ws]

    h, y = pl.pallas_call(
        _prep_body,
        grid=(_G,),
        in_specs=[_rows(5)] + wspecs(w_prep + w_msg),
        out_specs=[_rows(DIM), _rows(DIM)],
        out_shape=[jax.ShapeDtypeStruct((N, DIM), f32)] * 2,
    )(x, *w_prep, *w_msg)

    dagx = None
    for d in range(DEPTH):
        agg2, cnt2, dagx = _edge_sc(
            y, src_p, dst_p, maskf[d], z8, z1, xp8, ptr_i)
        cnt3 = cnt2.reshape(NC, R_TAB, 1)
        h, y = pl.pallas_call(
            _upd_body,
            grid=(_G,),
            in_specs=[
                pl.BlockSpec((1, _B, DIM), lambda i: (0, i, 0)),
                pl.BlockSpec((1, _B, DIM), lambda i: (1, i, 0)),
                pl.BlockSpec((1, _B, 1), lambda i: (0, i, 0)),
                pl.BlockSpec((1, _B, 1), lambda i: (1, i, 0)),
                _rows(DIM),
            ] + wspecs(w_upd + w_msg),
            out_specs=[_rows(DIM), _rows(DIM)],
            out_shape=[jax.ShapeDtypeStruct((N, DIM), f32)] * 2,
        )(agg2, agg2, cnt3, cnt3, h, *w_upd, *w_msg)

    w1 = params['dag_msg'][0][0]
    w_dag_split = ([w1[:5], w1[5:], w_dagm[1]] + w_dagm[2:])
    dsum, glob = pl.pallas_call(
        _dagsum_body,
        grid=(_G,),
        in_specs=[_rows(5), _rows(DIM), _full((1, NUM_DAGS)), _full((1, NUM_DAGS))]
        + wspecs(w_dag_split + w_glob),
        out_specs=[_full((NUM_DAGS, DIM)), _full((1, DIM))],
        out_shape=[jax.ShapeDtypeStruct((NUM_DAGS, DIM), f32),
                   jax.ShapeDtypeStruct((1, DIM), f32)],
    )(x, h, plo, phi, *w_dag_split, *w_glob)

    wn1 = params['node_score'][0][0]
    w_ns_split = ([wn1[:5], wn1[5:13], wn1[13:21], wn1[21:29], w_ns[1]]
                  + w_ns[2:])
    ns = pl.pallas_call(
        _nscore_body,
        grid=(_G,),
        in_specs=[_rows(5), _rows(DIM), _full((1, NUM_DAGS)), _full((1, NUM_DAGS)),
                  _full((NUM_DAGS, DIM)), _full((1, DIM))] + wspecs(w_ns_split),
        out_specs=[_rows(1)],
        out_shape=[jax.ShapeDtypeStruct((N, 1), f32)],
    )(x, h, plo, phi, dsum, glob, *w_ns_split)[0]

    wd1 = params['dag_score'][0][0]
    w_ds_split = ([wd1[:3], wd1[3:11], wd1[11:19], wd1[19:20].reshape(1, -1),
                   w_ds[1]] + w_ds[2:])
    dagf = dagx[:, :3]
    dsc = pl.pallas_call(
        _dscore_body,
        grid=(1,),
        in_specs=[_full((NUM_DAGS, 3)), _full((NUM_DAGS, DIM)), _full((1, DIM)),
                  _full((NEXEC, 1))] + wspecs(w_ds_split),
        out_specs=[_full((NUM_DAGS * NEXEC, 1))],
        out_shape=[jax.ShapeDtypeStruct((NUM_DAGS * NEXEC, 1), f32)],
    )(dagf, dsum, glob, ea, *w_ds_split)[0]

    return ns.reshape(N), dsc.reshape(NUM_DAGS, NEXEC)


# speedup vs baseline: 8.4800x; 1.4913x over previous
import functools

import jax
import jax.numpy as jnp
from jax import lax
from jax.experimental import pallas as pl
from jax.experimental.pallas import tpu as pltpu
from jax.experimental.pallas import tpu_sc as plsc

N = 100000
E = 3200000
DIM = 8
DEPTH = 8
NUM_DAGS = 128
NEXEC = 50

NC = 2
NS = 16
NW = NC * NS
EPW = 102400
E_PAD = NW * EPW
CHUNK = 1024
NVEC = CHUNK // 16
SUB = 128
MAXSUB = CHUNK // SUB
NPAIR = EPW // (2 * CHUNK)
R_TAB = 100352
SLAB = R_TAB // NS
TRASH = R_TAB - 8

_mesh = plsc.VectorSubcoreMesh(
    core_axis_name="c", subcore_axis_name="s", num_cores=NC, num_subcores=NS)


@functools.partial(
    pl.kernel,
    out_type=(
        jax.ShapeDtypeStruct((NC, R_TAB, DIM), jnp.float32),
        jax.ShapeDtypeStruct((NC, R_TAB), jnp.float32),
        jax.ShapeDtypeStruct((NUM_DAGS, 8), jnp.float32),
    ),
    mesh=_mesh,
    compiler_params=pltpu.CompilerParams(use_tc_tiling_on_sc=False,
                                         needs_layout_passes=False),
    scratch_types=[
        pltpu.VMEM_SHARED((R_TAB, DIM), jnp.float32),
        pltpu.VMEM_SHARED((R_TAB,), jnp.float32),
        pltpu.VMEM((CHUNK,), jnp.int32),
        pltpu.VMEM((CHUNK,), jnp.int32),
        pltpu.VMEM((CHUNK,), jnp.float32),
        pltpu.VMEM((CHUNK,), jnp.int32),
        pltpu.VMEM((CHUNK,), jnp.int32),
        pltpu.VMEM((CHUNK,), jnp.float32),
        pltpu.VMEM((CHUNK + SUB,), jnp.int32),
        pltpu.VMEM((CHUNK + SUB,), jnp.int32),
        pltpu.VMEM((MAXSUB, SUB), jnp.int32),
        pltpu.VMEM((MAXSUB, SUB), jnp.int32),
        pltpu.VMEM((CHUNK, DIM), jnp.float32),
        pltpu.VMEM((CHUNK, DIM), jnp.float32),
        pltpu.VMEM((SUB,), jnp.float32),
        pltpu.VMEM((NUM_DAGS,), jnp.int32),
        pltpu.VMEM((NUM_DAGS, 8), jnp.float32),
        pltpu.SemaphoreType.DMA,
        pltpu.SemaphoreType.DMA,
        pltpu.SemaphoreType.DMA,
        pltpu.SemaphoreType.DMA,
    ],
)
def _edge_sc(y_hbm, src_hbm, dst_hbm, mask_hbm, z8_hbm, z1_hbm, xp_hbm, ptr_hbm,
             agg_out, cnt_out, dagx_out,
             agg_sh, cnt_sh, src0, dst0, mk0, src1, dst1, mk1, psrc, pdst,
             sidx0, sidx1, rows0, rows1, ones_v, ptr_v, dagx_v,
             isem, gsem, ssem, csem):
    c = lax.axis_index("c")
    s = lax.axis_index("s")

    pltpu.sync_copy(z8_hbm, agg_sh.at[pl.ds(s * SLAB, SLAB)])
    pltpu.sync_copy(z1_hbm, cnt_sh.at[pl.ds(s * SLAB, SLAB)])
    for i in range(SUB // 16):
        ones_v[pl.ds(i * 16, 16)] = jnp.ones((16,), jnp.float32)
    plsc.subcore_barrier()

    base = (c * NS + s) * EPW
    trash = 100008 + s * 8

    def fire_in(ch, sv, dv, mv):
        off = base + ch * CHUNK
        pltpu.async_copy(src_hbm.at[pl.ds(off, CHUNK)], sv, isem)
        pltpu.async_copy(dst_hbm.at[pl.ds(off, CHUNK)], dv, isem)
        pltpu.async_copy(mask_hbm.at[pl.ds(off, CHUNK)], mv, isem)

    def drain_in(ch, sv, dv, mv):
        off = base + ch * CHUNK
        pltpu.make_async_copy(src_hbm.at[pl.ds(off, CHUNK)], sv, isem).wait()
        pltpu.make_async_copy(dst_hbm.at[pl.ds(off, CHUNK)], dv, isem).wait()
        pltpu.make_async_copy(mask_hbm.at[pl.ds(off, CHUNK)], mv, isem).wait()

    def compress(sv, dv, mv, sidx):
        vpr = SUB // 16
        def cpy(i, _):
            sl = pl.ds(i * 16, 16)
            m = mv[sl] > 0.0
            pdst[sl] = dv[sl]
            sidx[i // vpr, pl.ds((i % vpr) * 16, 16)] = jnp.where(
                m, sv[sl], jnp.zeros((16,), jnp.int32) + trash)
            return 0
        lax.fori_loop(0, NVEC, cpy, 0)
        return jnp.int32(MAXSUB)

    def fire_gather(nsub, rows):
        for j in range(MAXSUB):
            @pl.when(j < nsub)
            def _():
                pltpu.async_copy(y_hbm.at[pdst.at[pl.ds(j * SUB, SUB)]],
                                 rows.at[pl.ds(j * SUB, SUB)], gsem)

    def drain_gather(nsub, rows):
        for j in range(MAXSUB):
            @pl.when(j < nsub)
            def _():
                pltpu.make_async_copy(y_hbm.at[pdst.at[pl.ds(j * SUB, SUB)]],
                                      rows.at[pl.ds(j * SUB, SUB)], gsem).wait()

    def fire_scatter(nsub, rows, sidx):
        for j in range(MAXSUB):
            @pl.when(j < nsub)
            def _():
                pltpu.async_copy(rows.at[pl.ds(j * SUB, SUB)],
                                 agg_sh.at[sidx.at[j]], ssem, add=True)
                pltpu.async_copy(ones_v, cnt_sh.at[sidx.at[j]], csem, add=True)

    def drain_scatter(nsub, rows, sidx):
        for j in range(MAXSUB):
            @pl.when(j < nsub)
            def _():
                pltpu.make_async_copy(rows.at[pl.ds(j * SUB, SUB)],
                                      agg_sh.at[sidx.at[j]], ssem).wait()
                pltpu.make_async_copy(ones_v, cnt_sh.at[sidx.at[j]], csem).wait()

    fire_in(0, src0, dst0, mk0)
    fire_in(1, src1, dst1, mk1)

    def pair_body(t, ns_prev):
        drain_in(2 * t, src0, dst0, mk0)
        ns_a = compress(src0, dst0, mk0, sidx0)
        fire_in(2 * t + 2, src0, dst0, mk0)
        fire_gather(ns_a, rows0)
        drain_scatter(ns_prev, rows1, sidx1)
        drain_gather(ns_a, rows0)
        fire_scatter(ns_a, rows0, sidx0)
        drain_in(2 * t + 1, src1, dst1, mk1)
        ns_b = compress(src1, dst1, mk1, sidx1)
        fire_in(2 * t + 3, src1, dst1, mk1)
        fire_gather(ns_b, rows1)
        drain_scatter(ns_a, rows0, sidx0)
        drain_gather(ns_b, rows1)
        fire_scatter(ns_b, rows1, sidx1)
        return ns_b

    ns_last = lax.fori_loop(0, NPAIR, pair_body, 0)
    drain_scatter(ns_last, rows1, sidx1)
    drain_in(NPAIR * 2, src0, dst0, mk0)
    drain_in(NPAIR * 2 + 1, src1, dst1, mk1)

    plsc.subcore_barrier()
    pltpu.sync_copy(agg_sh.at[pl.ds(s * SLAB, SLAB)],
                    agg_out.at[c, pl.ds(s * SLAB, SLAB)])
    pltpu.sync_copy(cnt_sh.at[pl.ds(s * SLAB, SLAB)],
                    cnt_out.at[c, pl.ds(s * SLAB, SLAB)])

    @pl.when((c == 0) & (s == 0))
    def _():
        pltpu.sync_copy(ptr_hbm.at[pl.ds(0, NUM_DAGS)], ptr_v)
        pltpu.async_copy(xp_hbm.at[ptr_v], dagx_v, gsem).wait()
        pltpu.sync_copy(dagx_v, dagx_out)


_B = 2000
_G = N // _B


def _leaky(v):
    return jnp.maximum(v, 0.2 * v)


def _mm(a, w):
    return lax.dot_general(a, w, (((1,), (0,)), ((), ())),
                           preferred_element_type=jnp.float32)


def _mlp3(w, a):
    a = _leaky(_mm(a, w[0]) + w[1])
    a = _leaky(_mm(a, w[2]) + w[3])
    return _mm(a, w[4]) + w[5]


def _rows(m):
    return pl.BlockSpec((_B, m), lambda i: (i, 0))


def _full(shape):
    return pl.BlockSpec(shape, lambda i: (0,) * len(shape))


def _prep_body(x_ref, *refs):
    w = [r[...] for r in refs[:12]]
    h_ref, y_ref = refs[12:]
    h = _mlp3(w[:6], x_ref[...])
    h_ref[...] = h
    y_ref[...] = _mlp3(w[6:], h)


def _upd_body(aggA, aggB, cntA, cntB, h_ref, *refs):
    w = [r[...] for r in refs[:12]]
    hn_ref, yn_ref = refs[12:]
    agg = aggA[0] + aggB[0]
    cnt = cntA[0] + cntB[0]
    u = _mlp3(w[:6], agg)
    hn = h_ref[...] + jnp.where(cnt > 0.0, u, 0.0)
    hn_ref[...] = hn
    yn_ref[...] = _mlp3(w[6:], hn)


def _seg_onehot(plo_ref, phi_ref):
    i = pl.program_id(0)
    rowid = i * _B + lax.broadcasted_iota(jnp.int32, (_B, 1), 0)
    return ((rowid >= plo_ref[...]) & (rowid < phi_ref[...])).astype(jnp.float32)


def _dagsum_body(x_ref, h_ref, plo_ref, phi_ref, *refs):
    w = [r[...] for r in refs[:13]]
    dag_ref, glob_ref = refs[13:]
    i = pl.program_id(0)
    oh = _seg_onehot(plo_ref, phi_ref)
    z = _leaky(_mm(x_ref[...], w[0]) + _mm(h_ref[...], w[1]) + w[2])
    z = _leaky(_mm(z, w[3]) + w[4])
    z = _mm(z, w[5]) + w[6]
    part = lax.dot_general(oh, z, (((0,), (0,)), ((), ())),
                           preferred_element_type=jnp.float32)

    @pl.when(i == 0)
    def _():
        dag_ref[...] = part

    @pl.when(i > 0)
    def _():
        dag_ref[...] = dag_ref[...] + part

    @pl.when(i == _G - 1)
    def _():
        g = _mlp3(w[7:13], dag_ref[...])
        glob_ref[...] = jnp.sum(g, axis=0, keepdims=True)


def _nscore_body(x_ref, h_ref, plo_ref, phi_ref, dsum_ref, glob_ref, *refs):
    w = [r[...] for r in refs[:11]]
    ns_ref = refs[11]
    oh = _seg_onehot(plo_ref, phi_ref)
    drep = lax.dot_general(oh, dsum_ref[...], (((1,), (0,)), ((), ())),
                           preferred_element_type=jnp.float32)
    pre = (_mm(x_ref[...], w[0]) + _mm(h_ref[...], w[1]) + _mm(drep, w[2])
           + _mm(glob_ref[...], w[3]) + w[4])
    a = _leaky(pre)
    a = _leaky(_mm(a, w[5]) + w[6])
    a = _leaky(_mm(a, w[7]) + w[8])
    ns_ref[...] = _mm(a, w[9]) + w[10]


def _dscore_body(dagf_ref, dsum_ref, glob_ref, ea_ref, *refs):
    w = [r[...] for r in refs[:11]]
    out_ref = refs[11]
    m = (_mm(dagf_ref[...], w[0]) + _mm(dsum_ref[...], w[1])
         + _mm(glob_ref[...], w[2]) + w[4])
    e = ea_ref[...] * w[3]
    rows = NUM_DAGS * NEXEC
    q = ((lax.broadcasted_iota(jnp.int32, (rows, NUM_DAGS), 0) // NEXEC)
         == lax.broadcasted_iota(jnp.int32, (rows, NUM_DAGS), 1)).astype(jnp.float32)
    p = ((lax.broadcasted_iota(jnp.int32, (rows, NEXEC), 0) % NEXEC)
         == lax.broadcasted_iota(jnp.int32, (rows, NEXEC), 1)).astype(jnp.float32)
    a = _leaky(_mm(q, m) + _mm(p, e))
    a = _leaky(_mm(a, w[5]) + w[6])
    a = _leaky(_mm(a, w[7]) + w[8])
    out_ref[...] = _mm(a, w[9]) + w[10]


def _flat(layers):
    return [a for W, b in layers for a in (W, b.reshape(1, -1))]


def kernel(x, edge_index, edge_mask_batch, ptr, params):
    f32 = jnp.float32
    epad = E_PAD + CHUNK - E
    src_p = jnp.pad(edge_index[0], (0, epad)).astype(jnp.int32)
    dst_p = jnp.pad(edge_index[1], (0, epad)).astype(jnp.int32)
    maskf = jnp.pad(edge_mask_batch, ((0, 0), (0, epad))).astype(f32)
    xp8 = jnp.pad(x, ((0, 0), (0, 8 - x.shape[1])))
    ptr_i = ptr.astype(jnp.int32)
    z8 = jnp.zeros((SLAB, DIM), f32)
    z1 = jnp.zeros((SLAB,), f32)
    plo = ptr_i[:-1].reshape(1, NUM_DAGS)
    phi = ptr_i[1:].reshape(1, NUM_DAGS)
    ea = (jnp.arange(NEXEC, dtype=f32) / NEXEC).reshape(NEXEC, 1)

    w_prep = _flat(params['node_prep'])
    w_msg = _flat(params['node_msg'])
    w_upd = _flat(params['node_update'])
    w_dagm = _flat(params['dag_msg'])
    w_glob = _flat(params['glob_msg'])
    w_ns = _flat(params['node_score'])
    w_ds = _flat(params['dag_score'])

    wspecs = lambda ws: [_full(w.shape) for w in ws]

    h, y = pl.pallas_call(
        _prep_body,
        grid=(_G,),
        in_specs=[_rows(5)] + wspecs(w_prep + w_msg),
        out_specs=[_rows(DIM), _rows(DIM)],
        out_shape=[jax.ShapeDtypeStruct((N, DIM), f32)] * 2,
    )(x, *w_prep, *w_msg)

    dagx = None
    for d in range(DEPTH):
        agg2, cnt2, dagx = _edge_sc(
            y, src_p, dst_p, maskf[d], z8, z1, xp8, ptr_i)
        cnt3 = cnt2.reshape(NC, R_TAB, 1)
        h, y = pl.pallas_call(
            _upd_body,
            grid=(_G,),
            in_specs=[
                pl.BlockSpec((1, _B, DIM), lambda i: (0, i, 0)),
                pl.BlockSpec((1, _B, DIM), lambda i: (1, i, 0)),
                pl.BlockSpec((1, _B, 1), lambda i: (0, i, 0)),
                pl.BlockSpec((1, _B, 1), lambda i: (1, i, 0)),
                _rows(DIM),
            ] + wspecs(w_upd + w_msg),
            out_specs=[_rows(DIM), _rows(DIM)],
            out_shape=[jax.ShapeDtypeStruct((N, DIM), f32)] * 2,
        )(agg2, agg2, cnt3, cnt3, h, *w_upd, *w_msg)

    w1 = params['dag_msg'][0][0]
    w_dag_split = ([w1[:5], w1[5:], w_dagm[1]] + w_dagm[2:])
    dsum, glob = pl.pallas_call(
        _dagsum_body,
        grid=(_G,),
        in_specs=[_rows(5), _rows(DIM), _full((1, NUM_DAGS)), _full((1, NUM_DAGS))]
        + wspecs(w_dag_split + w_glob),
        out_specs=[_full((NUM_DAGS, DIM)), _full((1, DIM))],
        out_shape=[jax.ShapeDtypeStruct((NUM_DAGS, DIM), f32),
                   jax.ShapeDtypeStruct((1, DIM), f32)],
    )(x, h, plo, phi, *w_dag_split, *w_glob)

    wn1 = params['node_score'][0][0]
    w_ns_split = ([wn1[:5], wn1[5:13], wn1[13:21], wn1[21:29], w_ns[1]]
                  + w_ns[2:])
    ns = pl.pallas_call(
        _nscore_body,
        grid=(_G,),
        in_specs=[_rows(5), _rows(DIM), _full((1, NUM_DAGS)), _full((1, NUM_DAGS)),
                  _full((NUM_DAGS, DIM)), _full((1, DIM))] + wspecs(w_ns_split),
        out_specs=[_rows(1)],
        out_shape=[jax.ShapeDtypeStruct((N, 1), f32)],
    )(x, h, plo, phi, dsum, glob, *w_ns_split)[0]

    wd1 = params['dag_score'][0][0]
    w_ds_split = ([wd1[:3], wd1[3:11], wd1[11:19], wd1[19:20].reshape(1, -1),
                   w_ds[1]] + w_ds[2:])
    dagf = dagx[:, :3]
    dsc = pl.pallas_call(
        _dscore_body,
        grid=(1,),
        in_specs=[_full((NUM_DAGS, 3)), _full((NUM_DAGS, DIM)), _full((1, DIM)),
                  _full((NEXEC, 1))] + wspecs(w_ds_split),
        out_specs=[_full((NUM_DAGS * NEXEC, 1))],
        out_shape=[jax.ShapeDtypeStruct((NUM_DAGS * NEXEC, 1), f32)],
    )(dagf, dsum, glob, ea, *w_ds_split)[0]

    return ns.reshape(N), dsc.reshape(NUM_DAGS, NEXEC)
